# baseline (speedup 1.0000x reference)
import functools

import jax
import jax.numpy as jnp
from jax import lax
from jax.experimental import pallas as pl
from jax.experimental.pallas import tpu as pltpu
from jax.experimental.pallas import tpu_sc as plsc

_TIME_INTERVAL = 86400.0
_N_TIME_INTERVAL = 100
_LANES = 16
_REP = 8


@functools.partial(jax.jit, static_argnums=(2, 3, 4))
def _sc_lookup(table_rep, ts_flat, n_rows, row_len, emb):
    n = n_rows * row_len
    info = plsc.get_sparse_core_info()
    nc, ns = info.num_cores, info.num_subcores
    nw = nc * ns
    per_w = n // nw
    bs_w = n_rows // nw
    n_groups = per_w // _LANES
    stride = bs_w + 1
    net = emb // 8
    nbc = bs_w // _LANES

    mesh = plsc.VectorSubcoreMesh(core_axis_name="c", subcore_axis_name="s")

    @functools.partial(
        pl.kernel,
        mesh=mesh,
        out_type=jax.ShapeDtypeStruct((row_len, net, nw, 8, bs_w), jnp.float32),
        scratch_types=[
            pltpu.VMEM((8 + per_w + 8,), jnp.int32),
            pltpu.VMEM((table_rep.shape[0],), jnp.float32),
            pltpu.VMEM((row_len * stride + 8,), jnp.int32),
            pltpu.VMEM((2, net, 8, bs_w), jnp.float32),
            pltpu.SemaphoreType.DMA((2,)),
        ],
        compiler_params=pltpu.CompilerParams(
            use_tc_tiling_on_sc=False, needs_layout_passes=False),
    )
    def k(table_hbm, ts_hbm, out_hbm, ts_v, table_v, idx_v, plane_v, ssem):
        wid = lax.axis_index("s") * nc + lax.axis_index("c")
        base = wid * per_w
        pltpu.sync_copy(table_hbm, table_v)
        pltpu.sync_copy(ts_hbm.at[pl.ds(base, per_w)], ts_v.at[pl.ds(8, per_w)])
        iot = lax.iota(jnp.int32, _LANES)
        lane_rep = lax.rem(iot, jnp.int32(_REP))

        @plsc.parallel_loop(0, n_groups, unroll=2)
        def _phase1(g):
            off = g * _LANES
            cur = ts_v[pl.ds(off + 8, _LANES)]
            prev = ts_v[pl.ds(off + 7, _LANES)]
            diff = (cur - prev).astype(jnp.float32)
            t = diff / _TIME_INTERVAL * float(_N_TIME_INTERVAL)
            iv = t.astype(jnp.int32)
            iv = jnp.minimum(jnp.maximum(iv, 0), _N_TIME_INTERVAL)
            pos = off + iot
            rel = lax.rem(pos, jnp.int32(row_len))
            iv = jnp.where(rel == 0, 0, iv)
            brow = ((pos - rel).astype(jnp.float32) /
                    float(row_len)).astype(jnp.int32)
            plsc.store_scatter(idx_v, [rel * stride + brow], iv)

        vstep = (_N_TIME_INTERVAL + 1) * _REP

        def do_plane(l, s):
            ivs = []
            for c in range(nbc):
                raw = idx_v[pl.ds(l * stride + c * _LANES, _LANES)]
                ivs.append(raw * _REP + lane_rep)

            def te_body(te, carry):
                for es0 in range(0, 8, 2):
                    vs = []
                    for es in (es0, es0 + 1):
                        cbase = te * (8 * vstep) + es * vstep
                        vs.append([plsc.load_gather(table_v, [ivs[c] + cbase])
                                   for c in range(nbc)])
                    for k, es in enumerate((es0, es0 + 1)):
                        row_ref = plane_v.at[s, te, es]
                        for c in range(nbc):
                            row_ref[pl.ds(c * _LANES, _LANES)] = vs[k][c]
                pltpu.async_copy(
                    plane_v.at[s, te],
                    out_hbm.at[l, te, wid],
                    ssem.at[s],
                )
                return carry

            lax.fori_loop(0, net, te_body, 0)

        def wait_plane(s):
            for te in range(net):
                pltpu.make_async_copy(
                    plane_v.at[s, te],
                    out_hbm.at[0, te, wid],
                    ssem.at[s],
                ).wait()

        def body(i, carry):
            for s in (0, 1):
                l = i * 2 + s

                @pl.when(l >= 2)
                def _():
                    wait_plane(s)

                do_plane(l, s)
            return carry

        lax.fori_loop(0, row_len // 2, body, 0)
        wait_plane(0)
        wait_plane(1)

    return k(table_rep, ts_flat)


def kernel(inputs, timestamp, W, b):
    batch, max_len = timestamp.shape
    emb = W.shape[0]
    table_e_major = W + b[:, None]
    table_rep = jnp.broadcast_to(
        table_e_major[:, :, None], (emb, W.shape[1], _REP)).reshape(-1)
    out5 = _sc_lookup(table_rep, timestamp.reshape(batch * max_len),
                      batch, max_len, emb)
    return out5.transpose(2, 4, 0, 1, 3).reshape(batch, max_len, emb)


# speedup vs baseline: 1.0361x; 1.0361x over previous
import functools

import jax
import jax.numpy as jnp
from jax import lax
from jax.experimental import pallas as pl
from jax.experimental.pallas import tpu as pltpu
from jax.experimental.pallas import tpu_sc as plsc

_TIME_INTERVAL = 86400.0
_N_TIME_INTERVAL = 100
_LANES = 16
_REP = 8


@functools.partial(jax.jit, static_argnums=(2, 3, 4))
def _sc_lookup(table_rep, ts_flat, n_rows, row_len, emb):
    n = n_rows * row_len
    info = plsc.get_sparse_core_info()
    nc, ns = info.num_cores, info.num_subcores
    nw = nc * ns
    per_w = n // nw
    bs_w = n_rows // nw
    n_groups = per_w // _LANES
    stride = bs_w + 1
    net = emb // 8
    nbc = bs_w // _LANES

    mesh = plsc.VectorSubcoreMesh(core_axis_name="c", subcore_axis_name="s")

    @functools.partial(
        pl.kernel,
        mesh=mesh,
        out_type=jax.ShapeDtypeStruct((row_len, net, nw, 8, bs_w), jnp.float32),
        scratch_types=[
            pltpu.VMEM((8 + per_w + 8,), jnp.int32),
            pltpu.VMEM((table_rep.shape[0],), jnp.float32),
            pltpu.VMEM((row_len * stride + 8,), jnp.int32),
            pltpu.VMEM((2, net, 8, bs_w), jnp.float32),
            pltpu.SemaphoreType.DMA((2,)),
        ],
        compiler_params=pltpu.CompilerParams(
            use_tc_tiling_on_sc=False, needs_layout_passes=False),
    )
    def k(table_hbm, ts_hbm, out_hbm, ts_v, table_v, idx_v, plane_v, ssem):
        wid = lax.axis_index("s") * nc + lax.axis_index("c")
        base = wid * per_w
        pltpu.sync_copy(table_hbm, table_v)
        pltpu.sync_copy(ts_hbm.at[pl.ds(base, per_w)], ts_v.at[pl.ds(8, per_w)])
        iot = lax.iota(jnp.int32, _LANES)
        lane_rep = lax.rem(iot, jnp.int32(_REP))

        @plsc.parallel_loop(0, n_groups, unroll=2)
        def _phase1(g):
            off = g * _LANES
            cur = ts_v[pl.ds(off + 8, _LANES)]
            prev = ts_v[pl.ds(off + 7, _LANES)]
            diff = (cur - prev).astype(jnp.float32)
            t = diff / _TIME_INTERVAL * float(_N_TIME_INTERVAL)
            iv = t.astype(jnp.int32)
            iv = jnp.minimum(jnp.maximum(iv, 0), _N_TIME_INTERVAL)
            pos = off + iot
            rel = lax.rem(pos, jnp.int32(row_len))
            iv = jnp.where(rel == 0, 0, iv)
            brow = ((pos - rel).astype(jnp.float32) /
                    float(row_len)).astype(jnp.int32)
            plsc.store_scatter(idx_v, [rel * stride + brow], iv)

        vstep = (_N_TIME_INTERVAL + 1) * _REP

        def do_plane(l, s):
            ivs = []
            for c in range(nbc):
                raw = idx_v[pl.ds(l * stride + c * _LANES, _LANES)]
                ivs.append(raw * _REP + lane_rep)

            def te_body(te, carry):
                for es in range(8):
                    cbase = te * (8 * vstep) + es * vstep
                    row_ref = plane_v.at[s, te, es]
                    vs = [plsc.load_gather(table_v, [ivs[c] + cbase])
                          for c in range(nbc)]
                    for c in range(nbc):
                        row_ref[pl.ds(c * _LANES, _LANES)] = vs[c]
                pltpu.async_copy(
                    plane_v.at[s, te],
                    out_hbm.at[l, te, wid],
                    ssem.at[s],
                )
                return carry

            lax.fori_loop(0, net, te_body, 0)

        def wait_plane(s):
            for te in range(net):
                pltpu.make_async_copy(
                    plane_v.at[s, te],
                    out_hbm.at[0, te, wid],
                    ssem.at[s],
                ).wait()

        def body(i, carry):
            for s in (0, 1):
                l = i * 2 + s

                @pl.when(l >= 2)
                def _():
                    wait_plane(s)

                do_plane(l, s)
            return carry

        lax.fori_loop(0, row_len // 2, body, 0)
        wait_plane(0)
        wait_plane(1)

    return k(table_rep, ts_flat)


def kernel(inputs, timestamp, W, b):
    batch, max_len = timestamp.shape
    emb = W.shape[0]
    table_e_major = W + b[:, None]
    table_rep = jnp.broadcast_to(
        table_e_major[:, :, None], (emb, W.shape[1], _REP)).reshape(-1)
    out5 = _sc_lookup(table_rep, timestamp.reshape(batch * max_len),
                      batch, max_len, emb)
    return out5.transpose(2, 4, 0, 1, 3).reshape(batch, max_len, emb)
